# baseline (speedup 1.0000x reference)
import functools

import jax
import jax.numpy as jnp
from jax import lax
from jax.experimental import pallas as pl
from jax.experimental.pallas import tpu as pltpu
from jax.experimental.pallas import tpu_sc as plsc

NFEAT = 128
NIN = 256
NHID = 128
NNODES = 30720
NVIEW = 15360
NEDGES = 491520

NW = 32
EPW = NEDGES // NW
NBKT = 128
BSZ = NNODES // NBKT
NPASS = NBKT // NW
MAGIC = 34953
BSHIFT = 23
CHK = 64
ARENA = 12288
NEG = -1.0e30
PADVAL = NNODES << 15

@functools.lru_cache(maxsize=1)
def _mesh():
  return plsc.VectorSubcoreMesh(core_axis_name="c", subcore_axis_name="s")


def _tc1_body(x_ref, wrel_ref, brel_ref, p_ref, q_ref):
  xb = x_ref[...]
  wa = wrel_ref[:NFEAT, :]
  wb = wrel_ref[NFEAT:, :]
  q = jnp.dot(xb, wb, preferred_element_type=jnp.float32)
  p = jnp.dot(xb, wa - wb, preferred_element_type=jnp.float32) + brel_ref[...]
  p_ref[...] = p
  q_ref[...] = q.astype(jnp.bfloat16)


def _tc1(x, w_rel, b_rel2d):
  blk = 512
  grid = NNODES // blk
  return pl.pallas_call(
      _tc1_body,
      grid=(grid,),
      in_specs=[
          pl.BlockSpec((blk, NFEAT), lambda i: (i, 0)),
          pl.BlockSpec((2 * NFEAT, NIN), lambda i: (0, 0)),
          pl.BlockSpec((1, NIN), lambda i: (0, 0)),
      ],
      out_specs=[
          pl.BlockSpec((blk, NIN), lambda i: (i, 0)),
          pl.BlockSpec((blk, NIN), lambda i: (i, 0)),
      ],
      out_shape=[
          jax.ShapeDtypeStruct((NNODES, NIN), jnp.float32),
          jax.ShapeDtypeStruct((NNODES, NIN), jnp.bfloat16),
      ],
  )(x, w_rel, b_rel2d)


def _sc_bucket_body(c2c_hbm, lists_hbm, starts_hbm, ends_hbm,
                    src_v, dst_v, out_v, hist_v, cur_v, tmp_v, shf_v):
  cid = lax.axis_index("c")
  sid = lax.axis_index("s")
  wid = sid * 2 + cid
  base = wid * EPW

  pltpu.sync_copy(c2c_hbm.at[0, pl.ds(base, EPW)], src_v)
  pltpu.sync_copy(c2c_hbm.at[1, pl.ds(base, EPW)], dst_v)

  iota = lax.iota(jnp.int32, 16)
  im1 = jnp.maximum(iota - 1, 0)
  ip1 = jnp.minimum(iota + 1, 15)
  zero16 = jnp.zeros((16,), jnp.int32)
  for i in range(NBKT // 16):
    hist_v[pl.ds(i * 16, 16)] = zero16

  def _runs(sb):
    shf_v[pl.ds(0, 16)] = sb
    prev = plsc.load_gather(shf_v, [im1])
    is_start = (iota == 0) | (sb != prev)
    startpos = plsc.cummax(jnp.where(is_start, iota, 0))
    rank = iota - startpos
    shf_v[pl.ds(0, 16)] = jnp.where(is_start, 1, 0)
    nxt = plsc.load_gather(shf_v, [ip1])
    is_last = (iota == 15) | (nxt == 1)
    return rank, is_last

  def hist_step(i, carry):
    d = dst_v[pl.ds(i * 16, 16)]
    bkt = (d * MAGIC) >> BSHIFT
    sb, _ = plsc.sort_key_val(bkt, bkt)
    rank, is_last = _runs(sb)
    plsc.addupdate_scatter(hist_v, [sb], rank + 1, mask=is_last)
    return carry

  lax.fori_loop(0, EPW // 16, hist_step, 0)

  carry = jnp.int32(0)
  for i in range(NBKT // 16):
    h = hist_v[pl.ds(i * 16, 16)]
    inc = plsc.cumsum(h) + carry
    cur_v[pl.ds(i * 16, 16)] = inc - h
    tmp_v[pl.ds(i * 16, 16)] = inc
    carry = jnp.max(inc)

  pltpu.sync_copy(cur_v, starts_hbm.at[wid])
  pltpu.sync_copy(tmp_v, ends_hbm.at[wid])

  def place_step(i, carry):
    d = dst_v[pl.ds(i * 16, 16)]
    s = src_v[pl.ds(i * 16, 16)]
    bkt = (d * MAGIC) >> BSHIFT
    packed = d * 32768 + s
    sb, sp = plsc.sort_key_val(bkt, packed)
    rank, is_last = _runs(sb)
    woff = plsc.load_gather(cur_v, [sb]) + rank
    plsc.store_scatter(out_v, [woff], sp)
    plsc.addupdate_scatter(cur_v, [sb], rank + 1, mask=is_last)
    return carry

  lax.fori_loop(0, EPW // 16, place_step, 0)
  pltpu.sync_copy(out_v, lists_hbm.at[wid])


@functools.lru_cache(maxsize=1)
def _sc_bucket():
  return pl.kernel(
      _sc_bucket_body,
      out_type=[
          jax.ShapeDtypeStruct((NW, EPW), jnp.int32),
          jax.ShapeDtypeStruct((NW, NBKT), jnp.int32),
          jax.ShapeDtypeStruct((NW, NBKT), jnp.int32),
      ],
      mesh=_mesh(),
      scratch_types=[
          pltpu.VMEM((EPW,), jnp.int32),
          pltpu.VMEM((EPW,), jnp.int32),
          pltpu.VMEM((EPW,), jnp.int32),
          pltpu.VMEM((NBKT,), jnp.int32),
          pltpu.VMEM((NBKT,), jnp.int32),
          pltpu.VMEM((NBKT,), jnp.int32),
          pltpu.VMEM((16,), jnp.int32),
      ],
      compiler_params=pltpu.CompilerParams(needs_layout_passes=False),
  )


def _sc_segmax_body(q_hbm, lists_hbm, starts_hbm, ends_hbm, neg_hbm, qmax_hbm,
                    acc_v, row_v, arena_v, idx_v, dstl_v, st_v, en_v,
                    sem_e, sem_g):
  cid = lax.axis_index("c")
  sid = lax.axis_index("s")
  wid = sid * 2 + cid

  iota = lax.iota(jnp.int32, 16)

  def _extract(vec, j):
    return jnp.max(jnp.where(iota == j, vec, jnp.int32(-2147483647)))

  def _drain(nfly):
    def d(i, c):
      pltpu.make_async_copy(
          lists_hbm.at[pl.ds(0, CHK)], arena_v.at[pl.ds(0, CHK)], sem_e).wait()
      return c

    lax.fori_loop(0, nfly, d, 0)

  def _accum(pb):
    def group_body(g, carry):
      dl16 = dstl_v[pb, pl.ds(g * 16, 16)]
      ab16 = dl16 * NFEAT
      for k in range(16):
        dl = dl16[k]
        valid = (dl >= 0) & (dl < BSZ)
        abase = jnp.where(valid, ab16[k], BSZ * NFEAT)
        e = g * 16 + k
        NF = NIN // 32
        avals = [
            plsc.bitcast(acc_v[pl.ds(abase + f * 16, 16)], jnp.bfloat16)
            for f in range(NF)
        ]
        rvals = [
            plsc.bitcast(row_v[pb, e, pl.ds(f * 16, 16)], jnp.bfloat16)
            for f in range(NF)
        ]
        for f in range(NF):
          acc_v[pl.ds(abase + f * 16, 16)] = plsc.bitcast(
              jnp.maximum(avals[f], rvals[f]), jnp.int32)
      return carry

    lax.fori_loop(0, CHK // 16, group_body, 0)

  def _build_idx(cb, abase, b):
    for g in range(CHK // 16):
      e = arena_v[pl.ds(abase + g * 16, 16)]
      s = jnp.minimum(e & 32767, NNODES - 1)
      d = (e >> 15) - b * BSZ
      idx_v[cb, pl.ds(g * 16, 16)] = s
      dstl_v[cb, pl.ds(g * 16, 16)] = d

  def _process(slots, b):
    nach = slots >> (CHK.bit_length() - 1)

    @pl.when(nach > 0)
    def _():
      _build_idx(0, 0, b)
      pltpu.async_copy(q_hbm.at[idx_v.at[0]], row_v.at[0], sem_g)

      def gb(c, carry):
        cb = c & 1

        @pl.when(c + 1 < nach)
        def _():
          nb = (c + 1) & 1
          _build_idx(nb, pl.multiple_of((c + 1) * CHK, 8), b)
          pltpu.async_copy(q_hbm.at[idx_v.at[nb]], row_v.at[nb], sem_g)

        pltpu.make_async_copy(q_hbm.at[idx_v.at[cb]], row_v.at[cb],
                              sem_g).wait()
        _accum(cb)
        return carry

      lax.fori_loop(0, nach, gb, 0)

  def pass_body(ps, carry):
    b = ps * NW + wid
    pltpu.sync_copy(neg_hbm, acc_v)
    pltpu.sync_copy(starts_hbm.at[b], st_v)
    pltpu.sync_copy(ends_hbm.at[b], en_v)

    def producer_body(p, carry2):
      apos, nfly = carry2
      half = jnp.where(p < 16, 0, 1)
      stv = jnp.where(half == 0, st_v[pl.ds(0, 16)], st_v[pl.ds(16, 16)])
      env = jnp.where(half == 0, en_v[pl.ds(0, 16)], en_v[pl.ds(16, 16)])
      lane = p & 15
      st = _extract(stv, lane)
      en = _extract(env, lane)
      ast = st & ~7
      n = en - ast
      nch = (n + CHK - 1) >> CHK.bit_length() - 1
      gbase = pl.multiple_of(p * EPW + ast, 8)

      def fill_chunk(j, carry3):
        apos2, nfly2 = carry3
        full = apos2 + CHK > ARENA

        @pl.when(full)
        def _():
          _drain(nfly2)
          _process(apos2, b)

        apos2 = jnp.where(full, 0, apos2)
        nfly2 = jnp.where(full, 0, nfly2)
        pltpu.async_copy(
            lists_hbm.at[pl.ds(pl.multiple_of(gbase + j * CHK, 8), CHK)],
            arena_v.at[pl.ds(pl.multiple_of(apos2, 8), CHK)], sem_e)
        return (apos2 + CHK, nfly2 + 1)

      return lax.fori_loop(0, nch, fill_chunk, (apos, nfly))

    apos, nfly = lax.fori_loop(0, NW, producer_body, (0, 0))

    @pl.when(apos > 0)
    def _():
      _drain(nfly)
      _process(apos, b)

    pltpu.sync_copy(
        acc_v.at[pl.ds(0, BSZ * NFEAT)],
        qmax_hbm.at[pl.ds(pl.multiple_of(b * BSZ * NFEAT, 8), BSZ * NFEAT)])
    return carry

  lax.fori_loop(0, NPASS, pass_body, 0)


@functools.lru_cache(maxsize=1)
def _sc_segmax():
  return pl.kernel(
      _sc_segmax_body,
      out_type=jax.ShapeDtypeStruct((NNODES * NFEAT,), jnp.int32),
      mesh=_mesh(),
      scratch_types=[
          pltpu.VMEM(((BSZ + 1) * NFEAT,), jnp.int32),
          pltpu.VMEM((2, CHK, NFEAT), jnp.int32),
          pltpu.VMEM((ARENA,), jnp.int32),
          pltpu.VMEM((2, CHK), jnp.int32),
          pltpu.VMEM((2, CHK), jnp.int32),
          pltpu.VMEM((32,), jnp.int32),
          pltpu.VMEM((32,), jnp.int32),
          pltpu.SemaphoreType.DMA,
          pltpu.SemaphoreType.DMA,
      ],
      compiler_params=pltpu.CompilerParams(needs_layout_passes=False),
  )


def _tc2_body(p_ref, qm_ref, w1_ref, b1_ref, w2_ref, b2_ref, out_ref):
  qm = qm_ref[...].astype(jnp.float32)
  agg0 = jnp.maximum(p_ref[:, 0, :] + qm[:, 0, :], 0.0)
  agg1 = jnp.maximum(p_ref[:, 1, :] + qm[:, 1, :], 0.0)
  pooled = jnp.maximum(agg0, agg1)
  h = jnp.maximum(
      jnp.dot(pooled, w1_ref[...], preferred_element_type=jnp.float32)
      + b1_ref[...], 0.0)
  logits = (jnp.dot(h, w2_ref[...], preferred_element_type=jnp.float32)
            + b2_ref[...])
  m = jnp.max(logits, axis=1, keepdims=True)
  e = jnp.exp(logits - m)
  out_ref[...] = e / jnp.sum(e, axis=1, keepdims=True)


def _tc2(p3, qm3, w1, b1_2d, w2, b2_2d):
  blk = 512
  grid = NVIEW // blk
  return pl.pallas_call(
      _tc2_body,
      grid=(grid,),
      in_specs=[
          pl.BlockSpec((blk, 2, NIN), lambda i: (i, 0, 0)),
          pl.BlockSpec((blk, 2, NIN), lambda i: (i, 0, 0)),
          pl.BlockSpec((NIN, NHID), lambda i: (0, 0)),
          pl.BlockSpec((1, NHID), lambda i: (0, 0)),
          pl.BlockSpec((NHID, 2), lambda i: (0, 0)),
          pl.BlockSpec((1, 2), lambda i: (0, 0)),
      ],
      out_specs=pl.BlockSpec((blk, 2), lambda i: (i, 0)),
      out_shape=jax.ShapeDtypeStruct((NVIEW, 2), jnp.float32),
  )(p3, qm3, w1, b1_2d, w2, b2_2d)


def kernel(x1, x2, edge_index, batch, c2c_index, W_rel, b_rel, W1, b1, W2, b2):
  del edge_index, batch
  x = jnp.concatenate(
      [x1.reshape(-1, 30, NFEAT), x2.reshape(-1, 30, NFEAT)], axis=1
  ).reshape(-1, NFEAT)

  p_full, q_full = _tc1(x, W_rel, b_rel.reshape(1, -1))

  lists, starts, ends = _sc_bucket()(c2c_index)
  lists_flat = jnp.concatenate(
      [lists.reshape(-1), jnp.full((CHK,), PADVAL, jnp.int32)])
  starts_t = starts.T.reshape(NBKT, NW)
  ends_t = ends.T.reshape(NBKT, NW)
  negc = lax.bitcast_convert_type(
      jnp.full(((BSZ + 1) * NFEAT, 2), NEG, jnp.bfloat16), jnp.int32)

  q_i32 = lax.bitcast_convert_type(
      q_full.reshape(NNODES, NFEAT, 2), jnp.int32)
  qmax = _sc_segmax()(q_i32, lists_flat, starts_t, ends_t, negc)
  qmax = lax.bitcast_convert_type(
      qmax.reshape(NNODES, NFEAT), jnp.bfloat16).reshape(NNODES, NIN)

  return _tc2(
      p_full.reshape(NVIEW, 2, NIN), qmax.reshape(NVIEW, 2, NIN),
      W1, b1.reshape(1, -1), W2, b2.reshape(1, -1))


# speedup vs baseline: 3.8242x; 3.8242x over previous
import functools

import jax
import jax.numpy as jnp
from jax import lax
from jax.experimental import pallas as pl
from jax.experimental.pallas import tpu as pltpu
from jax.experimental.pallas import tpu_sc as plsc

NFEAT = 128
NIN = 256
NHID = 128
NNODES = 30720
NVIEW = 15360
NEDGES = 491520

NW = 32
EPW = NEDGES // NW
NBKT = 128
BSZ = NNODES // NBKT
NPASS = NBKT // NW
MAGIC = 34953
BSHIFT = 23
CHK = 64
ARENA = 12288
NEG = -1.0e30
PADVAL = NNODES << 15

@functools.lru_cache(maxsize=1)
def _mesh():
  return plsc.VectorSubcoreMesh(core_axis_name="c", subcore_axis_name="s")


def _tc1_body(x_ref, wrel_ref, brel_ref, p_ref, q_ref):
  xb = x_ref[...]
  wa = wrel_ref[:NFEAT, :]
  wb = wrel_ref[NFEAT:, :]
  q = jnp.dot(xb, wb, preferred_element_type=jnp.float32)
  p = jnp.dot(xb, wa - wb, preferred_element_type=jnp.float32) + brel_ref[...]
  p_ref[...] = p
  q_ref[...] = q


def _tc1(x, w_rel, b_rel2d):
  blk = 512
  grid = NNODES // blk
  return pl.pallas_call(
      _tc1_body,
      grid=(grid,),
      in_specs=[
          pl.BlockSpec((blk, NFEAT), lambda i: (i, 0)),
          pl.BlockSpec((2 * NFEAT, NIN), lambda i: (0, 0)),
          pl.BlockSpec((1, NIN), lambda i: (0, 0)),
      ],
      out_specs=[
          pl.BlockSpec((blk, NIN), lambda i: (i, 0)),
          pl.BlockSpec((blk, NIN), lambda i: (i, 0)),
      ],
      out_shape=[
          jax.ShapeDtypeStruct((NNODES, NIN), jnp.float32),
          jax.ShapeDtypeStruct((NNODES, NIN), jnp.float32),
      ],
  )(x, w_rel, b_rel2d)


def _sc_bucket_body(c2c_hbm, lists_hbm, starts_hbm, ends_hbm,
                    src_v, dst_v, out_v, hist_v, cur_v, tmp_v, shf_v):
  cid = lax.axis_index("c")
  sid = lax.axis_index("s")
  wid = sid * 2 + cid
  base = wid * EPW

  pltpu.sync_copy(c2c_hbm.at[0, pl.ds(base, EPW)], src_v)
  pltpu.sync_copy(c2c_hbm.at[1, pl.ds(base, EPW)], dst_v)

  iota = lax.iota(jnp.int32, 16)
  im1 = jnp.maximum(iota - 1, 0)
  ip1 = jnp.minimum(iota + 1, 15)
  zero16 = jnp.zeros((16,), jnp.int32)
  for i in range(NBKT // 16):
    hist_v[pl.ds(i * 16, 16)] = zero16

  def _runs(sb):
    shf_v[pl.ds(0, 16)] = sb
    prev = plsc.load_gather(shf_v, [im1])
    is_start = (iota == 0) | (sb != prev)
    startpos = plsc.cummax(jnp.where(is_start, iota, 0))
    rank = iota - startpos
    shf_v[pl.ds(0, 16)] = jnp.where(is_start, 1, 0)
    nxt = plsc.load_gather(shf_v, [ip1])
    is_last = (iota == 15) | (nxt == 1)
    return rank, is_last

  def hist_step(i, carry):
    d = dst_v[pl.ds(i * 16, 16)]
    bkt = (d * MAGIC) >> BSHIFT
    sb, _ = plsc.sort_key_val(bkt, bkt)
    rank, is_last = _runs(sb)
    plsc.addupdate_scatter(hist_v, [sb], rank + 1, mask=is_last)
    return carry

  lax.fori_loop(0, EPW // 16, hist_step, 0)

  carry = jnp.int32(0)
  for i in range(NBKT // 16):
    h = hist_v[pl.ds(i * 16, 16)]
    inc = plsc.cumsum(h) + carry
    cur_v[pl.ds(i * 16, 16)] = inc - h
    tmp_v[pl.ds(i * 16, 16)] = inc
    carry = jnp.max(inc)

  pltpu.sync_copy(cur_v, starts_hbm.at[wid])
  pltpu.sync_copy(tmp_v, ends_hbm.at[wid])

  def place_step(i, carry):
    d = dst_v[pl.ds(i * 16, 16)]
    s = src_v[pl.ds(i * 16, 16)]
    bkt = (d * MAGIC) >> BSHIFT
    packed = d * 32768 + s
    sb, sp = plsc.sort_key_val(bkt, packed)
    rank, is_last = _runs(sb)
    woff = plsc.load_gather(cur_v, [sb]) + rank
    plsc.store_scatter(out_v, [woff], sp)
    plsc.addupdate_scatter(cur_v, [sb], rank + 1, mask=is_last)
    return carry

  lax.fori_loop(0, EPW // 16, place_step, 0)
  pltpu.sync_copy(out_v, lists_hbm.at[wid])


@functools.lru_cache(maxsize=1)
def _sc_bucket():
  return pl.kernel(
      _sc_bucket_body,
      out_type=[
          jax.ShapeDtypeStruct((NW, EPW), jnp.int32),
          jax.ShapeDtypeStruct((NW, NBKT), jnp.int32),
          jax.ShapeDtypeStruct((NW, NBKT), jnp.int32),
      ],
      mesh=_mesh(),
      scratch_types=[
          pltpu.VMEM((EPW,), jnp.int32),
          pltpu.VMEM((EPW,), jnp.int32),
          pltpu.VMEM((EPW,), jnp.int32),
          pltpu.VMEM((NBKT,), jnp.int32),
          pltpu.VMEM((NBKT,), jnp.int32),
          pltpu.VMEM((NBKT,), jnp.int32),
          pltpu.VMEM((16,), jnp.int32),
      ],
      compiler_params=pltpu.CompilerParams(needs_layout_passes=False),
  )


def _sc_segmax_body(q_hbm, lists_hbm, starts_hbm, ends_hbm, neg_hbm, qmax_hbm,
                    acc_v, row_v, arena_v, idx_v, dstl_v, st_v, en_v,
                    sem_e, sem_g):
  cid = lax.axis_index("c")
  sid = lax.axis_index("s")
  wid = sid * 2 + cid

  iota = lax.iota(jnp.int32, 16)

  def _extract(vec, j):
    return jnp.max(jnp.where(iota == j, vec, jnp.int32(-2147483647)))

  def _drain(nfly):
    def d(i, c):
      pltpu.make_async_copy(
          lists_hbm.at[pl.ds(0, CHK)], arena_v.at[pl.ds(0, CHK)], sem_e).wait()
      return c

    lax.fori_loop(0, nfly, d, 0)

  def _accum(pb):
    def group_body(g, carry):
      dl16 = dstl_v[pb, pl.ds(g * 16, 16)]
      ab16 = dl16 * NIN
      for k in range(16):
        dl = dl16[k]
        valid = (dl >= 0) & (dl < BSZ)
        abase = jnp.where(valid, ab16[k], BSZ * NIN)
        e = g * 16 + k
        NF = NIN // 16
        avals = [acc_v[pl.ds(abase + f * 16, 16)] for f in range(NF)]
        rvals = [row_v[pb, e, pl.ds(f * 16, 16)] for f in range(NF)]
        for f in range(NF):
          acc_v[pl.ds(abase + f * 16, 16)] = jnp.maximum(avals[f], rvals[f])
      return carry

    lax.fori_loop(0, CHK // 16, group_body, 0)

  def _build_idx(cb, abase, b):
    for g in range(CHK // 16):
      e = arena_v[pl.ds(abase + g * 16, 16)]
      s = jnp.minimum(e & 32767, NNODES - 1)
      d = (e >> 15) - b * BSZ
      idx_v[cb, pl.ds(g * 16, 16)] = s
      dstl_v[cb, pl.ds(g * 16, 16)] = d

  def _process(slots, b):
    nach = slots >> (CHK.bit_length() - 1)

    @pl.when(nach > 0)
    def _():
      _build_idx(0, 0, b)
      pltpu.async_copy(q_hbm.at[idx_v.at[0]], row_v.at[0], sem_g)

      def gb(c, carry):
        cb = c & 1

        @pl.when(c + 1 < nach)
        def _():
          nb = (c + 1) & 1
          _build_idx(nb, pl.multiple_of((c + 1) * CHK, 8), b)
          pltpu.async_copy(q_hbm.at[idx_v.at[nb]], row_v.at[nb], sem_g)

        pltpu.make_async_copy(q_hbm.at[idx_v.at[cb]], row_v.at[cb],
                              sem_g).wait()
        _accum(cb)
        return carry

      lax.fori_loop(0, nach, gb, 0)

  def pass_body(ps, carry):
    b = ps * NW + wid
    pltpu.sync_copy(neg_hbm, acc_v)
    pltpu.sync_copy(starts_hbm.at[b], st_v)
    pltpu.sync_copy(ends_hbm.at[b], en_v)

    def producer_body(p, carry2):
      apos, nfly = carry2
      half = jnp.where(p < 16, 0, 1)
      stv = jnp.where(half == 0, st_v[pl.ds(0, 16)], st_v[pl.ds(16, 16)])
      env = jnp.where(half == 0, en_v[pl.ds(0, 16)], en_v[pl.ds(16, 16)])
      lane = p & 15
      st = _extract(stv, lane)
      en = _extract(env, lane)
      ast = st & ~7
      n = en - ast
      nch = (n + CHK - 1) >> CHK.bit_length() - 1
      gbase = pl.multiple_of(p * EPW + ast, 8)

      def fill_chunk(j, carry3):
        apos2, nfly2 = carry3
        full = apos2 + CHK > ARENA

        @pl.when(full)
        def _():
          _drain(nfly2)
          _process(apos2, b)

        apos2 = jnp.where(full, 0, apos2)
        nfly2 = jnp.where(full, 0, nfly2)
        pltpu.async_copy(
            lists_hbm.at[pl.ds(pl.multiple_of(gbase + j * CHK, 8), CHK)],
            arena_v.at[pl.ds(pl.multiple_of(apos2, 8), CHK)], sem_e)
        return (apos2 + CHK, nfly2 + 1)

      return lax.fori_loop(0, nch, fill_chunk, (apos, nfly))

    apos, nfly = lax.fori_loop(0, NW, producer_body, (0, 0))

    @pl.when(apos > 0)
    def _():
      _drain(nfly)
      _process(apos, b)

    pltpu.sync_copy(
        acc_v.at[pl.ds(0, BSZ * NIN)],
        qmax_hbm.at[pl.ds(pl.multiple_of(b * BSZ * NIN, 8), BSZ * NIN)])
    return carry

  lax.fori_loop(0, NPASS, pass_body, 0)


@functools.lru_cache(maxsize=1)
def _sc_segmax():
  return pl.kernel(
      _sc_segmax_body,
      out_type=jax.ShapeDtypeStruct((NNODES * NIN,), jnp.float32),
      mesh=_mesh(),
      scratch_types=[
          pltpu.VMEM(((BSZ + 1) * NIN,), jnp.float32),
          pltpu.VMEM((2, CHK, NIN), jnp.float32),
          pltpu.VMEM((ARENA,), jnp.int32),
          pltpu.VMEM((2, CHK), jnp.int32),
          pltpu.VMEM((2, CHK), jnp.int32),
          pltpu.VMEM((32,), jnp.int32),
          pltpu.VMEM((32,), jnp.int32),
          pltpu.SemaphoreType.DMA,
          pltpu.SemaphoreType.DMA,
      ],
      compiler_params=pltpu.CompilerParams(needs_layout_passes=False),
  )


def _tc2_body(p_ref, qm_ref, w1_ref, b1_ref, w2_ref, b2_ref, out_ref):
  qm = qm_ref[...].astype(jnp.float32)
  agg0 = jnp.maximum(p_ref[:, 0, :] + qm[:, 0, :], 0.0)
  agg1 = jnp.maximum(p_ref[:, 1, :] + qm[:, 1, :], 0.0)
  pooled = jnp.maximum(agg0, agg1)
  h = jnp.maximum(
      jnp.dot(pooled, w1_ref[...], preferred_element_type=jnp.float32)
      + b1_ref[...], 0.0)
  logits = (jnp.dot(h, w2_ref[...], preferred_element_type=jnp.float32)
            + b2_ref[...])
  m = jnp.max(logits, axis=1, keepdims=True)
  e = jnp.exp(logits - m)
  out_ref[...] = e / jnp.sum(e, axis=1, keepdims=True)


def _tc2(p3, qm3, w1, b1_2d, w2, b2_2d):
  blk = 512
  grid = NVIEW // blk
  return pl.pallas_call(
      _tc2_body,
      grid=(grid,),
      in_specs=[
          pl.BlockSpec((blk, 2, NIN), lambda i: (i, 0, 0)),
          pl.BlockSpec((blk, 2, NIN), lambda i: (i, 0, 0)),
          pl.BlockSpec((NIN, NHID), lambda i: (0, 0)),
          pl.BlockSpec((1, NHID), lambda i: (0, 0)),
          pl.BlockSpec((NHID, 2), lambda i: (0, 0)),
          pl.BlockSpec((1, 2), lambda i: (0, 0)),
      ],
      out_specs=pl.BlockSpec((blk, 2), lambda i: (i, 0)),
      out_shape=jax.ShapeDtypeStruct((NVIEW, 2), jnp.float32),
  )(p3, qm3, w1, b1_2d, w2, b2_2d)


def kernel(x1, x2, edge_index, batch, c2c_index, W_rel, b_rel, W1, b1, W2, b2):
  del edge_index, batch
  x = jnp.concatenate(
      [x1.reshape(-1, 30, NFEAT), x2.reshape(-1, 30, NFEAT)], axis=1
  ).reshape(-1, NFEAT)

  p_full, q_full = _tc1(x, W_rel, b_rel.reshape(1, -1))

  lists, starts, ends = _sc_bucket()(c2c_index)
  lists_flat = jnp.concatenate(
      [lists.reshape(-1), jnp.full((CHK,), PADVAL, jnp.int32)])
  starts_t = starts.T.reshape(NBKT, NW)
  ends_t = ends.T.reshape(NBKT, NW)
  negc = jnp.full(((BSZ + 1) * NIN,), NEG, jnp.float32)

  qmax = _sc_segmax()(q_full, lists_flat, starts_t, ends_t, negc)
  qmax = qmax.reshape(NNODES, NIN)

  return _tc2(
      p_full.reshape(NVIEW, 2, NIN), qmax.reshape(NVIEW, 2, NIN),
      W1, b1.reshape(1, -1), W2, b2.reshape(1, -1))
